# baseline (speedup 1.0000x reference)
import functools

import jax
import jax.numpy as jnp
from jax import lax
from jax.experimental import pallas as pl
from jax.experimental.pallas import tpu as pltpu
from jax.experimental.pallas import tpu_sc as plsc

_D = 64
_NC = 2
_NS = 16
_NW = _NC * _NS
_CHUNK = 128
_CPG = 2
_GROUP = _CHUNK * _CPG
_NBANK = 5
_AHEAD = 4
_TCHUNKS = 5


@functools.lru_cache(maxsize=None)
def _make_gather(b_total: int):
  b_per_w = b_total // _NW
  n_chunk = b_per_w // _CHUNK
  n_group = n_chunk // _CPG
  assert n_group % _NBANK == 0
  n_outer = n_group // _NBANK
  mesh = plsc.VectorSubcoreMesh(core_axis_name="c", subcore_axis_name="s")

  bank_types = [pltpu.VMEM((_GROUP, _D), jnp.float32) for _ in range(_NBANK)]
  gsem_types = [pltpu.SemaphoreType.DMA for _ in range(_NBANK)]
  ssem_types = [pltpu.SemaphoreType.DMA for _ in range(_NBANK)]

  @functools.partial(
      pl.kernel,
      mesh=mesh,
      out_type=jax.ShapeDtypeStruct((b_total, _D), jnp.float32),
      scratch_types=[pltpu.VMEM((n_chunk, _CHUNK), jnp.int32)]
      + bank_types + gsem_types + ssem_types,
      compiler_params=pltpu.CompilerParams(use_tc_tiling_on_sc=False),
  )
  def gather_kernel(table_hbm, idx_hbm, out_hbm, idx_v, *scratch):
    banks = scratch[:_NBANK]
    gsems = scratch[_NBANK:2 * _NBANK]
    ssems = scratch[2 * _NBANK:]
    wid = lax.axis_index("s") * _NC + lax.axis_index("c")
    pltpu.sync_copy(idx_hbm.at[wid], idx_v)
    base = wid * b_per_w

    def fire(g, b):
      for c in range(_CPG):
        pltpu.async_copy(
            table_hbm.at[idx_v.at[_CPG * g + c]],
            banks[b].at[pl.ds(c * _CHUNK, _CHUNK)],
            gsems[b],
        )

    def wait_gathers(b):
      pltpu.make_async_copy(
          table_hbm.at[pl.ds(0, _GROUP)], banks[b], gsems[b]).wait()

    def store(g, b):
      pltpu.async_copy(
          banks[b], out_hbm.at[pl.ds(base + g * _GROUP, _GROUP)], ssems[b])

    def wait_store(b):
      pltpu.make_async_copy(
          banks[b], out_hbm.at[pl.ds(base, _GROUP)], ssems[b]).wait()

    for g in range(_AHEAD):
      fire(g, g % _NBANK)

    def body(q, carry):
      for s in range(_NBANK):
        g = q * _NBANK + s
        b = s
        wait_gathers(b)
        store(g, b)
        nb = (s + _AHEAD) % _NBANK
        if s + _AHEAD < _NBANK:
          @pl.when(q > 0)
          def _():
            wait_store(nb)
            fire(g + _AHEAD, nb)

          @pl.when(q == 0)
          def _():
            fire(g + _AHEAD, nb)
        else:
          @pl.when(q * _NBANK + s + _AHEAD < n_group)
          def _():
            wait_store(nb)
            fire(g + _AHEAD, nb)
      return carry

    lax.fori_loop(0, n_outer, body, 0)

    for b in range(_NBANK):
      wait_store(b)

  return gather_kernel


def kernel(embed, timestep):
  b, t = timestep.shape
  b_total = b * t
  embed_p = jnp.pad(embed, ((0, 0), (0, _D))).reshape(2 * embed.shape[0], _D)
  th = t // 2
  b_half = b * th
  gather = _make_gather(b_half)
  ts2 = timestep.astype(jnp.int32) * 2
  idx1 = ts2.T[:th].reshape(_NW, b_half // (_NW * _CHUNK), _CHUNK)
  idx2 = ts2[:, th:].reshape(_NW, b_half // (_NW * _CHUNK), _CHUNK)
  p1 = gather(embed_p, idx1)
  p2 = gather(embed_p, idx2)
  o1 = p1.reshape(th, b, _D).transpose(1, 0, 2)
  o2 = p2.reshape(b, th, _D)
  return jnp.concatenate([o1, o2], axis=1)


# speedup vs baseline: 1.1149x; 1.1149x over previous
import functools

import jax
import jax.numpy as jnp
from jax import lax
from jax.experimental import pallas as pl
from jax.experimental.pallas import tpu as pltpu
from jax.experimental.pallas import tpu_sc as plsc

_D = 64
_NC = 2
_NS = 16
_NW = _NC * _NS
_CHUNK = 128
_CPG = 2
_GROUP = _CHUNK * _CPG
_NBANK = 5
_AHEAD = 4


@functools.lru_cache(maxsize=None)
def _make_gather(b_total: int):
  b_per_w = b_total // _NW
  n_chunk = b_per_w // _CHUNK
  n_group = n_chunk // _CPG
  assert n_group % _NBANK == 0
  n_outer = n_group // _NBANK
  mesh = plsc.VectorSubcoreMesh(core_axis_name="c", subcore_axis_name="s")

  bank_types = [pltpu.VMEM((_GROUP, _D), jnp.float32) for _ in range(_NBANK)]
  gsem_types = [pltpu.SemaphoreType.DMA for _ in range(_NBANK)]
  ssem_types = [pltpu.SemaphoreType.DMA for _ in range(_NBANK)]

  @functools.partial(
      pl.kernel,
      mesh=mesh,
      out_type=jax.ShapeDtypeStruct((b_total, _D), jnp.float32),
      scratch_types=[pltpu.VMEM((n_chunk, _CHUNK), jnp.int32)]
      + bank_types + gsem_types + ssem_types,
      compiler_params=pltpu.CompilerParams(use_tc_tiling_on_sc=False),
  )
  def gather_kernel(table_hbm, idx_hbm, out_hbm, idx_v, *scratch):
    banks = scratch[:_NBANK]
    gsems = scratch[_NBANK:2 * _NBANK]
    ssems = scratch[2 * _NBANK:]
    wid = lax.axis_index("s") * _NC + lax.axis_index("c")
    pltpu.sync_copy(idx_hbm.at[wid], idx_v)
    base = wid * b_per_w

    def fire(g, b):
      for c in range(_CPG):
        pltpu.async_copy(
            table_hbm.at[idx_v.at[_CPG * g + c]],
            banks[b].at[pl.ds(c * _CHUNK, _CHUNK)],
            gsems[b],
        )

    def wait_gathers(b):
      pltpu.make_async_copy(
          table_hbm.at[pl.ds(0, _GROUP)], banks[b], gsems[b]).wait()

    def store(g, b):
      pltpu.async_copy(
          banks[b], out_hbm.at[pl.ds(base + g * _GROUP, _GROUP)], ssems[b])

    def wait_store(b):
      pltpu.make_async_copy(
          banks[b], out_hbm.at[pl.ds(base, _GROUP)], ssems[b]).wait()

    for g in range(_AHEAD):
      fire(g, g % _NBANK)

    def body(q, carry):
      for s in range(_NBANK):
        g = q * _NBANK + s
        b = s
        wait_gathers(b)
        store(g, b)
        nb = (s + _AHEAD) % _NBANK
        if s + _AHEAD < _NBANK:
          @pl.when(q > 0)
          def _():
            wait_store(nb)
            fire(g + _AHEAD, nb)

          @pl.when(q == 0)
          def _():
            fire(g + _AHEAD, nb)
        else:
          @pl.when(q * _NBANK + s + _AHEAD < n_group)
          def _():
            wait_store(nb)
            fire(g + _AHEAD, nb)
      return carry

    lax.fori_loop(0, n_outer, body, 0)

    for b in range(_NBANK):
      wait_store(b)

  return gather_kernel


def kernel(embed, timestep):
  b, t = timestep.shape
  b_total = b * t
  embed_p = jnp.pad(embed, ((0, 0), (0, _D))).reshape(2 * embed.shape[0], _D)
  idx = (timestep.T.astype(jnp.int32) * 2).reshape(
      _NW, b_total // (_NW * _CHUNK), _CHUNK)
  out = _make_gather(b_total)(embed_p, idx)
  return out.reshape(t, b, _D).transpose(1, 0, 2)
